# baseline (speedup 1.0000x reference)
import functools

import jax
import jax.numpy as jnp
from jax import lax
from jax.experimental import pallas as pl
from jax.experimental.pallas import tpu as pltpu

_EPS = 1e-12


def _round_up(v, n):
    return (v + n - 1) // n * n


def _core_kernel(lab_ref, x_ref, w_hbm, o_hbm, xn_s, lab_s,
                 w_buf, o_buf, w_sem, o_sem,
                 *, s, m, ch, F, Bp):
    j = pl.program_id(0)
    base = 2 * j

    def w_copy(c, slot):
        return pltpu.make_async_copy(
            w_hbm.at[pl.ds((base + c) * ch, ch), :],
            w_buf.at[slot],
            w_sem.at[slot])

    def o_copy(c, slot):
        return pltpu.make_async_copy(
            o_buf.at[slot],
            o_hbm.at[:, pl.ds((base + c) * ch, ch)],
            o_sem.at[slot])

    w_copy(0, 0).start()
    w_copy(1, 1).start()
    x = x_ref[...]
    sx = jnp.sum(x * x, axis=1, keepdims=True)
    inv_nx = lax.rsqrt(jnp.maximum(sx, _EPS * _EPS)) * s
    xn_s[...] = (x * inv_nx).astype(jnp.bfloat16)
    lab_s[...] = lab_ref[...].reshape(Bp, 1)

    def chunk_res(c, slot):
        w = w_buf[slot]
        sw = jnp.sum(w * w, axis=1, keepdims=True)
        inv_nw = lax.rsqrt(jnp.maximum(sw, _EPS * _EPS))
        wn = (w * inv_nw).astype(jnp.bfloat16)
        raw = lax.dot_general(
            xn_s[...], wn,
            dimension_numbers=(((1,), (1,)), ((), ())),
            preferred_element_type=jnp.float32)
        class_ids = lax.broadcasted_iota(jnp.int32, raw.shape, 1)
        labels = lab_s[...] - (base + c) * ch
        return jnp.where(class_ids == labels, raw - (s * m), raw)

    w_copy(0, 0).wait()
    o_buf[0] = chunk_res(0, 0)
    w_copy(1, 1).wait()
    o_copy(0, 0).start()
    o_buf[1] = chunk_res(1, 1)
    o_copy(1, 1).start()
    o_copy(0, 0).wait()
    o_copy(1, 1).wait()


def kernel(x, W, label, s=30.0, m=0.35):
    B, F = x.shape
    C, F2 = W.shape
    assert F == F2

    Bp = _round_up(B, 8)
    Cp = _round_up(C, 512)
    ch = Cp // 4
    x_p = x if Bp == B else jnp.pad(x, ((0, Bp - B), (0, 0)))
    W_p = W if Cp == C else jnp.pad(W, ((0, Cp - C), (0, 0)))
    lab = label.astype(jnp.int32).reshape(1, B)
    lab_p = lab if Bp == B else jnp.pad(lab, ((0, 0), (0, Bp - B)),
                                        constant_values=-1)

    out = pl.pallas_call(
        functools.partial(_core_kernel, s=s, m=m, ch=ch, F=F, Bp=Bp),
        out_shape=jax.ShapeDtypeStruct((Bp, Cp), jnp.float32),
        grid=(2,),
        in_specs=[
            pl.BlockSpec((1, Bp), lambda j: (0, 0)),
            pl.BlockSpec((Bp, F), lambda j: (0, 0)),
            pl.BlockSpec(memory_space=pl.ANY),
        ],
        out_specs=pl.BlockSpec(memory_space=pl.ANY),
        scratch_shapes=[
            pltpu.VMEM((Bp, F), jnp.bfloat16),
            pltpu.VMEM((Bp, 1), jnp.int32),
            pltpu.VMEM((2, ch, F), jnp.float32),
            pltpu.VMEM((2, Bp, ch), jnp.float32),
            pltpu.SemaphoreType.DMA((2,)),
            pltpu.SemaphoreType.DMA((2,)),
        ],
        compiler_params=pltpu.CompilerParams(
            dimension_semantics=("parallel",),
            vmem_limit_bytes=58 * 1024 * 1024,
        ),
    )(lab_p, x_p, W_p)
    return out[:B, :C]


# speedup vs baseline: 1.0828x; 1.0828x over previous
import functools

import jax
import jax.numpy as jnp
from jax import lax
from jax.experimental import pallas as pl
from jax.experimental.pallas import tpu as pltpu

_EPS = 1e-12


def _round_up(v, n):
    return (v + n - 1) // n * n


def _core_kernel(lab_ref, x_ref, w_ref, o_hbm, o_buf, o_sem,
                 *, s, m, tile_c, ch):
    j = pl.program_id(0)

    def o_copy(half):
        return pltpu.make_async_copy(
            o_buf.at[half],
            o_hbm.at[:, pl.ds(j * tile_c + half * ch, ch)],
            o_sem.at[half])

    x = x_ref[...]
    sx = jnp.sum(x * x, axis=1, keepdims=True)
    inv_nx = lax.rsqrt(jnp.maximum(sx, _EPS * _EPS)) * s
    xn = (x * inv_nx).astype(jnp.bfloat16)
    lab_col = lab_ref[...].reshape(-1, 1)

    def half_res(half):
        w = w_ref[pl.ds(half * ch, ch), :]
        sw = jnp.sum(w * w, axis=1, keepdims=True)
        inv_nw = lax.rsqrt(jnp.maximum(sw, _EPS * _EPS))
        wn = (w * inv_nw).astype(jnp.bfloat16)
        raw = lax.dot_general(
            xn, wn,
            dimension_numbers=(((1,), (1,)), ((), ())),
            preferred_element_type=jnp.float32)
        class_ids = lax.broadcasted_iota(jnp.int32, raw.shape, 1)
        labels = lab_col - (j * tile_c + half * ch)
        return jnp.where(class_ids == labels, raw - (s * m), raw)

    o_buf[0] = half_res(0)
    o_copy(0).start()
    o_buf[1] = half_res(1)
    o_copy(1).start()
    o_copy(0).wait()
    o_copy(1).wait()


def kernel(x, W, label, s=30.0, m=0.35, tile_c=5120):
    B, F = x.shape
    C, F2 = W.shape
    assert F == F2

    tc = tile_c if C >= tile_c else _round_up(C, 256)
    ch = tc // 2
    Bp = _round_up(B, 8)
    Cp = _round_up(C, tc)
    x_p = x if Bp == B else jnp.pad(x, ((0, Bp - B), (0, 0)))
    W_p = W if Cp == C else jnp.pad(W, ((0, Cp - C), (0, 0)))
    lab = label.astype(jnp.int32).reshape(1, B)
    lab_p = lab if Bp == B else jnp.pad(lab, ((0, 0), (0, Bp - B)),
                                        constant_values=-1)

    out = pl.pallas_call(
        functools.partial(_core_kernel, s=s, m=m, tile_c=tc, ch=ch),
        out_shape=jax.ShapeDtypeStruct((Bp, Cp), jnp.float32),
        grid=(Cp // tc,),
        in_specs=[
            pl.BlockSpec((1, Bp), lambda j: (0, 0)),
            pl.BlockSpec((Bp, F), lambda j: (0, 0)),
            pl.BlockSpec((tc, F), lambda j: (j, 0)),
        ],
        out_specs=pl.BlockSpec(memory_space=pl.ANY),
        scratch_shapes=[
            pltpu.VMEM((2, Bp, ch), jnp.float32),
            pltpu.SemaphoreType.DMA((2,)),
        ],
        compiler_params=pltpu.CompilerParams(
            dimension_semantics=("parallel",),
            vmem_limit_bytes=58 * 1024 * 1024,
        ),
    )(lab_p, x_p, W_p)
    return out[:B, :C]


# speedup vs baseline: 1.1521x; 1.0640x over previous
import functools

import jax
import jax.numpy as jnp
from jax import lax
from jax.experimental import pallas as pl
from jax.experimental.pallas import tpu as pltpu

_EPS = 1e-12


def _round_up(v, n):
    return (v + n - 1) // n * n


def _cosface_fused_kernel(lab_ref, x_ref, w_ref, o_ref, *, s, m, tile_c):
    x = x_ref[...]
    sx = jnp.sum(x * x, axis=1, keepdims=True)
    inv_nx = lax.rsqrt(jnp.maximum(sx, _EPS * _EPS)) * s
    xn = (x * inv_nx).astype(jnp.bfloat16)

    w = w_ref[...]
    sw = jnp.sum(w * w, axis=1, keepdims=True)
    inv_nw = lax.rsqrt(jnp.maximum(sw, _EPS * _EPS))
    wn = (w * inv_nw).astype(jnp.bfloat16)

    raw = lax.dot_general(
        xn, wn,
        dimension_numbers=(((1,), (1,)), ((), ())),
        preferred_element_type=jnp.float32)

    col0 = pl.program_id(0) * tile_c
    class_ids = lax.broadcasted_iota(jnp.int32, raw.shape, 1)
    labels = lab_ref[...].reshape(-1, 1) - col0
    o_ref[...] = jnp.where(class_ids == labels, raw - (s * m), raw)


def kernel(x, W, label, s=30.0, m=0.35, tile_c=5120):
    B, F = x.shape
    C, F2 = W.shape
    assert F == F2

    tc = tile_c if C >= tile_c else _round_up(C, 128)
    Bp = _round_up(B, 8)
    Cp = _round_up(C, tc)
    x_p = x if Bp == B else jnp.pad(x, ((0, Bp - B), (0, 0)))
    W_p = W if Cp == C else jnp.pad(W, ((0, Cp - C), (0, 0)))
    lab = label.astype(jnp.int32).reshape(1, B)
    lab_p = lab if Bp == B else jnp.pad(lab, ((0, 0), (0, Bp - B)),
                                        constant_values=-1)

    out = pl.pallas_call(
        functools.partial(_cosface_fused_kernel, s=s, m=m, tile_c=tc),
        out_shape=jax.ShapeDtypeStruct((Bp, Cp), jnp.float32),
        grid=(Cp // tc,),
        in_specs=[
            pl.BlockSpec((1, Bp), lambda j: (0, 0)),
            pl.BlockSpec((Bp, F), lambda j: (0, 0)),
            pl.BlockSpec((tc, F), lambda j: (j, 0)),
        ],
        out_specs=pl.BlockSpec((Bp, tc), lambda j: (0, j)),
        compiler_params=pltpu.CompilerParams(
            dimension_semantics=("parallel",),
            vmem_limit_bytes=58 * 1024 * 1024,
        ),
    )(lab_p, x_p, W_p)
    return out[:B, :C]
